# baseline (speedup 1.0000x reference)
import functools

import numpy as np
import jax
import jax.numpy as jnp
from jax import lax
from jax.experimental import pallas as pl
from jax.experimental.pallas import tpu as pltpu
from jax.experimental.pallas import tpu_sc as plsc

V = 2048
B = 64
K = 16
DEPTH = 11
NC = 2
NS = 16
NBIG = 7
NCHUNKS = 127
NSMALL = 15
LN2 = 0.6931471805599453
NEG_HALF_LOG_2PI = -0.9189385332046727

_f32 = jnp.float32
_i32 = jnp.int32


def _bitrev_table(n: int) -> np.ndarray:
    bits = n.bit_length() - 1
    idx = np.arange(n)
    rev = np.zeros(n, dtype=np.int32)
    for b in range(bits):
        rev |= ((idx >> b) & 1) << (bits - 1 - b)
    return rev.astype(np.int32)


def _rev11(v):
    r = jnp.zeros_like(v)
    for b in range(11):
        r = jnp.bitwise_or(r, jnp.left_shift(jnp.bitwise_and(jnp.right_shift(v, b), 1), 10 - b))
    return r


def _tree_max(vs):
    while len(vs) > 1:
        vs = [jnp.maximum(vs[i], vs[i + 1]) for i in range(0, len(vs) - 1, 2)] \
            + ([vs[-1]] if len(vs) % 2 else [])
    return vs[0]


def _tree_sum(vs):
    while len(vs) > 1:
        vs = [vs[i] + vs[i + 1] for i in range(0, len(vs) - 1, 2)] \
            + ([vs[-1]] if len(vs) % 2 else [])
    return vs[0]


def _ln(v):
    i = lax.bitcast_convert_type(v, _i32)
    ef = (jnp.right_shift(i, 23) - 127).astype(_f32)
    m = lax.bitcast_convert_type(
        jnp.bitwise_or(jnp.bitwise_and(i, 0x007FFFFF), 0x3F800000), _f32)
    r = (m - 1.0) / (m + 1.0)
    r2 = r * r
    p = 1.0 + r2 * (np.float32(1.0 / 3.0) + r2 * (np.float32(0.2) + r2 * np.float32(1.0 / 7.0)))
    return ef * np.float32(LN2) + 2.0 * r * p


def _sc_body(xp, muf, lsf, wflat, wsmf, brtab, out, wnbig, wnsm,
             h0, h1, hS0, hS1, xb, mub, lsb, btb, wnb, wpa, wpb, wsa, wsb,
             wsmb, e0b, e1b, obuf, pa0, pa1, msem, wsem, psem):
    cid = lax.axis_index("c")
    sid = lax.axis_index("s")
    wid = cid * NS + sid
    b0 = wid * 2
    iot = lax.iota(_i32, 16)

    pltpu.sync_copy(xp.at[b0], xb.at[0])
    pltpu.sync_copy(xp.at[b0 + 1], xb.at[1])
    pltpu.sync_copy(brtab, btb)

    def _mls_start(seg, par):
        pltpu.async_copy(muf.at[pl.ds(seg * 4096, 4096)], mub.at[par], msem.at[par])
        pltpu.async_copy(lsf.at[pl.ds(seg * 4096, 4096)], lsb.at[par], msem.at[par])

    def _mls_wait(seg, par):
        pltpu.make_async_copy(muf.at[pl.ds(seg * 4096, 4096)], mub.at[par], msem.at[par]).wait()
        pltpu.make_async_copy(lsf.at[pl.ds(seg * 4096, 4096)], lsb.at[par], msem.at[par]).wait()

    _mls_start(0, 0)
    _mls_start(1, 1)

    scope = jax.named_scope
    def _prep_chunk(gi):
        l = ((gi >= 64).astype(_i32) + (gi >= 96) + (gi >= 112)
             + (gi >= 120) + (gi >= 124) + (gi >= 126))
        base = jnp.int32(128) - (jnp.int32(128) >> l)
        c = gi - base
        s16 = c * 16 + iot
        f = jnp.right_shift(_rev11(s16), 1 + l)
        rows = (jnp.int32(2048) - (jnp.int32(2048) >> l)) + f
        pltpu.async_copy(wflat.at[rows], wpa, psem).wait()

        def jbody(j, carry):
            cols = [jnp.broadcast_to(j * 16 + k, (16,)).astype(_i32)
                    for k in range(K)]
            m = _tree_max([plsc.load_gather(wpa, [iot, cols[k]])
                           for k in range(K)])
            es = [jnp.exp(plsc.load_gather(wpa, [iot, cols[k]]) - m)
                  for k in range(K)]
            inv = 1.0 / _tree_sum(es)
            for k in range(K):
                wpb[k, j, :] = es[k] * inv
            return carry

        lax.fori_loop(0, K, jbody, 0)
        pltpu.sync_copy(wpb, wnbig.at[cid, gi])

    def pbody(i, carry):
        gi = sid + NS * i

        @pl.when(gi < NCHUNKS)
        def _():
            _prep_chunk(gi)
        return carry

    with scope("wprep"):
        lax.fori_loop(0, 8, pbody, 0)

    @pl.when(sid < NSMALL)
    def _():
        s = sid
        g0 = s < 8
        g1 = s < 12
        g2 = s < 14
        bs = jnp.where(g0, 0, jnp.where(g1, 8, jnp.where(g2, 12, 14)))
        fl = s - bs
        r3 = (jnp.left_shift(jnp.bitwise_and(fl, 1), 2)
              | jnp.bitwise_and(fl, 2) | jnp.bitwise_and(jnp.right_shift(fl, 2), 1))
        r2 = jnp.left_shift(jnp.bitwise_and(fl, 1), 1) | jnp.right_shift(fl, 1)
        row = bs + jnp.where(g0, r3, jnp.where(g1, r2, fl))
        pltpu.async_copy(wsmf.at[row], wsa, psem).wait()
        m = _tree_max([plsc.load_gather(wsa, [iot * 16 + k]) for k in range(K)])
        es = [jnp.exp(plsc.load_gather(wsa, [iot * 16 + k]) - m)
              for k in range(K)]
        inv = 1.0 / _tree_sum(es)
        for k in range(K):
            wsb[k, :] = es[k] * inv
        pltpu.sync_copy(wsb, wnsm.at[cid, sid])

    inscope = jax.named_scope("input_layer")
    inscope.__enter__()
    for seg in range(8):
        par = seg % 2
        _mls_wait(seg, par)
        mu_s = mub.at[par]
        ls_s = lsb.at[par]

        def cbody(c, carry2, seg=seg, mu_s=mu_s, ls_s=ls_s):
            col = pl.multiple_of(seg * 256 + c * 16, 16)
            cbase = c * 256
            bc = btb[pl.ds(col, 16)]
            for xrow, hP, hS in ((0, h0, hS0), (1, h1, hS1)):
                xv = xb[xrow, pl.ds(col, 16)]
                ll = []
                for k in range(K):
                    idx = iot * 16 + (cbase + k)
                    muv = plsc.load_gather(mu_s, [idx])
                    lsv = plsc.load_gather(ls_s, [idx])
                    z = (xv - muv) * jnp.exp(-lsv)
                    ll.append(np.float32(NEG_HALF_LOG_2PI) - lsv - 0.5 * (z * z))
                s = _tree_max(ll)
                plsc.store_scatter(hS, [bc], s)
                for k in range(K):
                    plsc.store_scatter(hP, [bc + k * 2048], jnp.exp(ll[k] - s))
            return carry2

        lax.fori_loop(0, 16, cbody, 0)
        if seg + 2 < 8:
            _mls_start(seg + 2, par)

    inscope.__exit__(None, None, None)
    with jax.named_scope("barrier"):
        plsc.subcore_barrier()

    def _wn_start(gi, par):
        pltpu.async_copy(wnbig.at[cid, gi], wnb.at[par], wsem.at[par])

    def _wn_wait(gi, par):
        pltpu.make_async_copy(wnbig.at[cid, gi], wnb.at[par], wsem.at[par]).wait()

    for q in range(4):
        _wn_start(q, q)

    def chunk_body(gi, carry):
        l = ((gi >= 64).astype(_i32) + (gi >= 96) + (gi >= 112)
             + (gi >= 120) + (gi >= 124) + (gi >= 126))
        f2 = jnp.int32(1024) >> l
        base = jnp.int32(128) - (jnp.int32(128) >> l)
        c = gi - base
        par = lax.rem(gi, 4)
        lcol = pl.multiple_of(c * 16, 16)
        rcol = pl.multiple_of(f2 + c * 16, 16)
        _wn_wait(gi, par)
        rm0 = None
        rm1 = None
        for half in (0, 1):
            kr = range(half * 8, half * 8 + 8)
            e0 = [h0[pl.ds(pl.multiple_of(k * 2048 + lcol, 16), 16)]
                  * h0[pl.ds(pl.multiple_of(k * 2048 + rcol, 16), 16)]
                  for k in kr]
            e1 = [h1[pl.ds(pl.multiple_of(k * 2048 + lcol, 16), 16)]
                  * h1[pl.ds(pl.multiple_of(k * 2048 + rcol, 16), 16)]
                  for k in kr]
            for j in range(K):
                ws = [wnb[par, k, j, :] for k in kr]
                a = _tree_sum([e0[i] * ws[i] for i in range(8)])
                b = _tree_sum([e1[i] * ws[i] for i in range(8)])
                if half == 0:
                    pa0[pl.ds(j * 16, 16)] = a
                    pa1[pl.ds(j * 16, 16)] = b
                else:
                    a = a + pa0[pl.ds(j * 16, 16)]
                    b = b + pa1[pl.ds(j * 16, 16)]
                    h0[pl.ds(pl.multiple_of(j * 2048 + lcol, 16), 16)] = a
                    h1[pl.ds(pl.multiple_of(j * 2048 + lcol, 16), 16)] = b
                    rm0 = a if rm0 is None else jnp.maximum(rm0, a)
                    rm1 = b if rm1 is None else jnp.maximum(rm1, b)
        for hP, hS, rm in ((h0, hS0, rm0), (h1, hS1, rm1)):
            eb = jnp.right_shift(lax.bitcast_convert_type(rm, _i32), 23)
            sc = lax.bitcast_convert_type(jnp.left_shift(254 - eb, 23), _f32)
            for j in range(K):
                off = pl.ds(pl.multiple_of(j * 2048 + lcol, 16), 16)
                hP[off] = hP[off] * sc
            hS[pl.ds(lcol, 16)] = (hS[pl.ds(lcol, 16)] + hS[pl.ds(rcol, 16)]
                                   + (eb - 127).astype(_f32) * np.float32(LN2))
        nxt = gi + 4

        @pl.when(nxt < NCHUNKS)
        def _():
            _wn_start(nxt, par)
        return carry

    with jax.named_scope("chunks"):
        lax.fori_loop(0, NCHUNKS, chunk_body, 0)

    pltpu.sync_copy(wnsm.at[cid], wsmb)
    m0mask = iot == 0

    def tail_body(s, carry):
        g0 = s < 8
        g1 = s < 12
        g2 = s < 14
        bs = jnp.where(g0, 0, jnp.where(g1, 8, jnp.where(g2, 12, 14)))
        f2s = jnp.where(g0, 8, jnp.where(g1, 4, jnp.where(g2, 2, 1)))
        lcol = s - bs
        rcol = lcol + f2s
        lsp = jnp.broadcast_to(lcol, (16,)).astype(_i32)
        rsp = jnp.broadcast_to(rcol, (16,)).astype(_i32)
        lidx = iot * 2048 + lsp
        ridx = iot * 2048 + rsp
        e0b[...] = plsc.load_gather(h0, [lidx]) * plsc.load_gather(h0, [ridx])
        e1b[...] = plsc.load_gather(h1, [lidx]) * plsc.load_gather(h1, [ridx])
        kk = jnp.broadcast_to(0, (16,)).astype(_i32)
        w = wsmb[s, 0, :]
        a0 = plsc.load_gather(e0b, [kk]) * w
        a1 = plsc.load_gather(e1b, [kk]) * w
        for k in range(1, K):
            kk = jnp.broadcast_to(k, (16,)).astype(_i32)
            w = wsmb[s, k, :]
            a0 = a0 + plsc.load_gather(e0b, [kk]) * w
            a1 = a1 + plsc.load_gather(e1b, [kk]) * w
        mx0 = jnp.broadcast_to(jnp.max(a0), (16,))
        mx1 = jnp.broadcast_to(jnp.max(a1), (16,))
        eb0 = jnp.right_shift(lax.bitcast_convert_type(mx0, _i32), 23)
        eb1 = jnp.right_shift(lax.bitcast_convert_type(mx1, _i32), 23)
        sc0 = lax.bitcast_convert_type(jnp.left_shift(254 - eb0, 23), _f32)
        sc1 = lax.bitcast_convert_type(jnp.left_shift(254 - eb1, 23), _f32)
        plsc.store_scatter(h0, [lidx], a0 * sc0)
        plsc.store_scatter(h1, [lidx], a1 * sc1)
        s0 = (plsc.load_gather(hS0, [lsp]) + plsc.load_gather(hS0, [rsp])
              + (eb0 - 127).astype(_f32) * np.float32(LN2))
        s1 = (plsc.load_gather(hS1, [lsp]) + plsc.load_gather(hS1, [rsp])
              + (eb1 - 127).astype(_f32) * np.float32(LN2))
        plsc.store_scatter(hS0, [lsp], s0, mask=m0mask)
        plsc.store_scatter(hS1, [lsp], s1, mask=m0mask)
        return carry

    with jax.named_scope("tail"):
        lax.fori_loop(0, NSMALL, tail_body, 0)

    z16 = jnp.broadcast_to(0, (16,)).astype(_i32)
    obuf[...] = plsc.load_gather(hS0, [z16]) + _ln(plsc.load_gather(h0, [iot * 2048]))
    pltpu.sync_copy(obuf, out.at[b0])
    obuf[...] = plsc.load_gather(hS1, [z16]) + _ln(plsc.load_gather(h1, [iot * 2048]))
    pltpu.sync_copy(obuf, out.at[b0 + 1])


_sc_call = functools.partial(
    pl.kernel,
    out_type=(
        jax.ShapeDtypeStruct((B, 16), _f32),
        jax.ShapeDtypeStruct((NC, NCHUNKS, K, K, 16), _f32),
        jax.ShapeDtypeStruct((NC, NSMALL, K, K), _f32),
    ),
    mesh=plsc.VectorSubcoreMesh(
        core_axis_name="c", subcore_axis_name="s", num_cores=NC, num_subcores=NS),
    compiler_params=pltpu.CompilerParams(
        needs_layout_passes=False, use_tc_tiling_on_sc=False),
    scratch_types=(
        pltpu.VMEM((K * V,), _f32),
        pltpu.VMEM((K * V,), _f32),
        pltpu.VMEM((V,), _f32),
        pltpu.VMEM((V,), _f32),
        pltpu.VMEM((2, V), _f32),
        pltpu.VMEM((2, 4096), _f32),
        pltpu.VMEM((2, 4096), _f32),
        pltpu.VMEM((V,), _i32),
        pltpu.VMEM((4, K, K, 16), _f32),
        pltpu.VMEM((16, 256), _f32),
        pltpu.VMEM((K, K, 16), _f32),
        pltpu.VMEM((256,), _f32),
        pltpu.VMEM((K, K), _f32),
        pltpu.VMEM((NSMALL, K, K), _f32),
        pltpu.VMEM((16,), _f32),
        pltpu.VMEM((16,), _f32),
        pltpu.VMEM((16,), _f32),
        pltpu.VMEM((256,), _f32),
        pltpu.VMEM((256,), _f32),
        pltpu.SemaphoreType.DMA((2,)),
        pltpu.SemaphoreType.DMA((4,)),
        pltpu.SemaphoreType.DMA,
    ),
)(_sc_body)

_BRTAB_NP = _bitrev_table(V)


def kernel(x, mu, logsig, w00, w01, w02, w03, w04, w05, w06, w07, w08, w09, w10):
    ws = [w00, w01, w02, w03, w04, w05, w06, w07, w08, w09, w10]
    xp = x.reshape(B, V)
    muf = mu.reshape(V * K)
    lsf = logsig.reshape(V * K)
    wflat = jnp.concatenate([ws[l].reshape(-1, 256) for l in range(NBIG)], axis=0)
    wsm_parts = [ws[l].reshape(-1, 256) for l in range(NBIG, DEPTH - 1)]
    wsm_parts.append(jnp.broadcast_to(w10.reshape(1, 1, K), (1, K, K)).reshape(1, 256))
    wsmf = jnp.concatenate(wsm_parts, axis=0)
    out_wide, _, _ = _sc_call(xp, muf, lsf, wflat, wsmf, jnp.asarray(_BRTAB_NP))
    return out_wide[:, :1]


# speedup vs baseline: 1.0100x; 1.0100x over previous
import functools

import numpy as np
import jax
import jax.numpy as jnp
from jax import lax
from jax.experimental import pallas as pl
from jax.experimental.pallas import tpu as pltpu
from jax.experimental.pallas import tpu_sc as plsc

V = 2048
B = 64
K = 16
DEPTH = 11
NC = 2
NS = 16
NBIG = 7
NCHUNKS = 127
NSMALL = 15
LN2 = 0.6931471805599453
NEG_HALF_LOG_2PI = -0.9189385332046727

_f32 = jnp.float32
_i32 = jnp.int32


def _bitrev_table(n: int) -> np.ndarray:
    bits = n.bit_length() - 1
    idx = np.arange(n)
    rev = np.zeros(n, dtype=np.int32)
    for b in range(bits):
        rev |= ((idx >> b) & 1) << (bits - 1 - b)
    return rev.astype(np.int32)


def _rev11(v):
    r = jnp.zeros_like(v)
    for b in range(11):
        r = jnp.bitwise_or(r, jnp.left_shift(jnp.bitwise_and(jnp.right_shift(v, b), 1), 10 - b))
    return r


def _tree_max(vs):
    while len(vs) > 1:
        vs = [jnp.maximum(vs[i], vs[i + 1]) for i in range(0, len(vs) - 1, 2)] \
            + ([vs[-1]] if len(vs) % 2 else [])
    return vs[0]


def _tree_sum(vs):
    while len(vs) > 1:
        vs = [vs[i] + vs[i + 1] for i in range(0, len(vs) - 1, 2)] \
            + ([vs[-1]] if len(vs) % 2 else [])
    return vs[0]


def _ln(v):
    i = lax.bitcast_convert_type(v, _i32)
    ef = (jnp.right_shift(i, 23) - 127).astype(_f32)
    m = lax.bitcast_convert_type(
        jnp.bitwise_or(jnp.bitwise_and(i, 0x007FFFFF), 0x3F800000), _f32)
    r = (m - 1.0) / (m + 1.0)
    r2 = r * r
    p = 1.0 + r2 * (np.float32(1.0 / 3.0) + r2 * (np.float32(0.2) + r2 * np.float32(1.0 / 7.0)))
    return ef * np.float32(LN2) + 2.0 * r * p


def _sc_body(xp, muf, lsf, wflat, wsmf, brtab, out, wnbig, wnsm,
             h0, h1, hS0, hS1, xb, mub, lsb, btb, wnb, wpa, wpb, wsa, wsb,
             wsmb, e0b, e1b, obuf, pa0, pa1, msem, wsem, psem):
    cid = lax.axis_index("c")
    sid = lax.axis_index("s")
    wid = cid * NS + sid
    b0 = wid * 2
    iot = lax.iota(_i32, 16)

    pltpu.sync_copy(xp.at[b0], xb.at[0])
    pltpu.sync_copy(xp.at[b0 + 1], xb.at[1])
    pltpu.sync_copy(brtab, btb)

    def _mls_start(seg, par):
        pltpu.async_copy(muf.at[pl.ds(seg * 4096, 4096)], mub.at[par], msem.at[par])
        pltpu.async_copy(lsf.at[pl.ds(seg * 4096, 4096)], lsb.at[par], msem.at[par])

    def _mls_wait(seg, par):
        pltpu.make_async_copy(muf.at[pl.ds(seg * 4096, 4096)], mub.at[par], msem.at[par]).wait()
        pltpu.make_async_copy(lsf.at[pl.ds(seg * 4096, 4096)], lsb.at[par], msem.at[par]).wait()

    _mls_start(0, 0)
    _mls_start(1, 1)

    scope = jax.named_scope

    def _rows_of(gi):
        l = ((gi >= 64).astype(_i32) + (gi >= 96) + (gi >= 112)
             + (gi >= 120) + (gi >= 124) + (gi >= 126))
        base = jnp.int32(128) - (jnp.int32(128) >> l)
        c = gi - base
        s16 = c * 16 + iot
        f = jnp.right_shift(_rev11(s16), 1 + l)
        return (jnp.int32(2048) - (jnp.int32(2048) >> l)) + f

    def _prep_start(gi, pp):
        pltpu.async_copy(wflat.at[_rows_of(gi)], wpa.at[pp], psem.at[pp])

    def _prep_wait(gi, pp):
        pltpu.make_async_copy(wflat.at[_rows_of(gi)], wpa.at[pp], psem.at[pp]).wait()

    def _prep_chunk(gi, pp):
        ppv = jnp.broadcast_to(pp, (16,)).astype(_i32)

        def jbody(j, carry):
            cols = [jnp.broadcast_to(j * 16 + k, (16,)).astype(_i32)
                    for k in range(K)]
            m = _tree_max([plsc.load_gather(wpa, [ppv, iot, cols[k]])
                           for k in range(K)])
            es = [jnp.exp(plsc.load_gather(wpa, [ppv, iot, cols[k]]) - m)
                  for k in range(K)]
            inv = 1.0 / _tree_sum(es)
            for k in range(K):
                wpb[k, j, :] = es[k] * inv
            return carry

        lax.fori_loop(0, K, jbody, 0)
        pltpu.sync_copy(wpb, wnbig.at[cid, gi])

    @pl.when(sid < NCHUNKS)
    def _():
        _prep_start(sid, 0)

    @pl.when(sid + NS < NCHUNKS)
    def _():
        _prep_start(sid + NS, 1)

    def pbody(i, carry):
        gi = sid + NS * i
        pp = lax.rem(i, 2)

        @pl.when(gi < NCHUNKS)
        def _():
            _prep_wait(gi, pp)
            _prep_chunk(gi, pp)

        @pl.when(gi + 2 * NS < NCHUNKS)
        def _():
            _prep_start(gi + 2 * NS, pp)
        return carry

    with scope("wprep"):
        lax.fori_loop(0, 8, pbody, 0)

    @pl.when(sid < NSMALL)
    def _():
        s = sid
        g0 = s < 8
        g1 = s < 12
        g2 = s < 14
        bs = jnp.where(g0, 0, jnp.where(g1, 8, jnp.where(g2, 12, 14)))
        fl = s - bs
        r3 = (jnp.left_shift(jnp.bitwise_and(fl, 1), 2)
              | jnp.bitwise_and(fl, 2) | jnp.bitwise_and(jnp.right_shift(fl, 2), 1))
        r2 = jnp.left_shift(jnp.bitwise_and(fl, 1), 1) | jnp.right_shift(fl, 1)
        row = bs + jnp.where(g0, r3, jnp.where(g1, r2, fl))
        pltpu.async_copy(wsmf.at[row], wsa, psem.at[0]).wait()
        m = _tree_max([plsc.load_gather(wsa, [iot * 16 + k]) for k in range(K)])
        es = [jnp.exp(plsc.load_gather(wsa, [iot * 16 + k]) - m)
              for k in range(K)]
        inv = 1.0 / _tree_sum(es)
        for k in range(K):
            wsb[k, :] = es[k] * inv
        pltpu.sync_copy(wsb, wnsm.at[cid, sid])

    inscope = jax.named_scope("input_layer")
    inscope.__enter__()
    for seg in range(8):
        par = seg % 2
        _mls_wait(seg, par)
        mu_s = mub.at[par]
        ls_s = lsb.at[par]

        def cbody(c, carry2, seg=seg, mu_s=mu_s, ls_s=ls_s):
            col = pl.multiple_of(seg * 256 + c * 16, 16)
            cbase = c * 256
            bc = btb[pl.ds(col, 16)]
            xv0 = xb[0, pl.ds(col, 16)]
            xv1 = xb[1, pl.ds(col, 16)]
            for k in range(K):
                idx = iot * 16 + (cbase + k)
                muv = plsc.load_gather(mu_s, [idx])
                lsv = plsc.load_gather(ls_s, [idx])
                inv = jnp.exp(-lsv)
                ct = np.float32(NEG_HALF_LOG_2PI) - lsv
                z0 = (xv0 - muv) * inv
                z1 = (xv1 - muv) * inv
                plsc.store_scatter(h0, [bc + k * 2048], ct - 0.5 * (z0 * z0))
                plsc.store_scatter(h1, [bc + k * 2048], ct - 0.5 * (z1 * z1))
            return carry2

        lax.fori_loop(0, 16, cbody, 0)
        if seg + 2 < 8:
            _mls_start(seg + 2, par)

    inscope.__exit__(None, None, None)
    with jax.named_scope("barrier"):
        plsc.subcore_barrier()

    def _wn_start(gi, par):
        pltpu.async_copy(wnbig.at[cid, gi], wnb.at[par], wsem.at[par])

    def _wn_wait(gi, par):
        pltpu.make_async_copy(wnbig.at[cid, gi], wnb.at[par], wsem.at[par]).wait()

    for q in range(4):
        _wn_start(q, q)

    def chunk0_body(gi, carry):
        par = lax.rem(gi, 4)
        lcol = pl.multiple_of(gi * 16, 16)
        rcol = pl.multiple_of(1024 + gi * 16, 16)
        _wn_wait(gi, par)
        p0 = [h0[pl.ds(pl.multiple_of(k * 2048 + lcol, 16), 16)]
              + h0[pl.ds(pl.multiple_of(k * 2048 + rcol, 16), 16)]
              for k in range(K)]
        p1 = [h1[pl.ds(pl.multiple_of(k * 2048 + lcol, 16), 16)]
              + h1[pl.ds(pl.multiple_of(k * 2048 + rcol, 16), 16)]
              for k in range(K)]
        m0 = _tree_max(p0)
        m1 = _tree_max(p1)
        e0 = [jnp.exp(p0[k] - m0) for k in range(K)]
        e1 = [jnp.exp(p1[k] - m1) for k in range(K)]
        rm0 = None
        rm1 = None
        for j in range(K):
            w = wnb[par, 0, j, :]
            a0 = e0[0] * w
            a1 = e1[0] * w
            for k in range(1, K):
                w = wnb[par, k, j, :]
                a0 = a0 + e0[k] * w
                a1 = a1 + e1[k] * w
            h0[pl.ds(pl.multiple_of(j * 2048 + lcol, 16), 16)] = a0
            h1[pl.ds(pl.multiple_of(j * 2048 + lcol, 16), 16)] = a1
            rm0 = a0 if rm0 is None else jnp.maximum(rm0, a0)
            rm1 = a1 if rm1 is None else jnp.maximum(rm1, a1)
        for hP, hS, rm, m in ((h0, hS0, rm0, m0), (h1, hS1, rm1, m1)):
            eb = jnp.right_shift(lax.bitcast_convert_type(rm, _i32), 23)
            sc = lax.bitcast_convert_type(jnp.left_shift(254 - eb, 23), _f32)
            for j in range(K):
                off = pl.ds(pl.multiple_of(j * 2048 + lcol, 16), 16)
                hP[off] = hP[off] * sc
            hS[pl.ds(lcol, 16)] = m + (eb - 127).astype(_f32) * np.float32(LN2)
        nxt = gi + 4

        @pl.when(nxt < NCHUNKS)
        def _():
            _wn_start(nxt, par)
        return carry

    def chunk_body(gi, carry):
        l = ((gi >= 64).astype(_i32) + (gi >= 96) + (gi >= 112)
             + (gi >= 120) + (gi >= 124) + (gi >= 126))
        f2 = jnp.int32(1024) >> l
        base = jnp.int32(128) - (jnp.int32(128) >> l)
        c = gi - base
        par = lax.rem(gi, 4)
        lcol = pl.multiple_of(c * 16, 16)
        rcol = pl.multiple_of(f2 + c * 16, 16)
        _wn_wait(gi, par)
        e0 = [h0[pl.ds(pl.multiple_of(k * 2048 + lcol, 16), 16)]
              * h0[pl.ds(pl.multiple_of(k * 2048 + rcol, 16), 16)]
              for k in range(K)]
        e1 = [h1[pl.ds(pl.multiple_of(k * 2048 + lcol, 16), 16)]
              * h1[pl.ds(pl.multiple_of(k * 2048 + rcol, 16), 16)]
              for k in range(K)]
        rm0 = None
        rm1 = None
        for j in range(K):
            w = wnb[par, 0, j, :]
            a0 = e0[0] * w
            a1 = e1[0] * w
            for k in range(1, K):
                w = wnb[par, k, j, :]
                a0 = a0 + e0[k] * w
                a1 = a1 + e1[k] * w
            h0[pl.ds(pl.multiple_of(j * 2048 + lcol, 16), 16)] = a0
            h1[pl.ds(pl.multiple_of(j * 2048 + lcol, 16), 16)] = a1
            rm0 = a0 if rm0 is None else jnp.maximum(rm0, a0)
            rm1 = a1 if rm1 is None else jnp.maximum(rm1, a1)
        for hP, hS, rm in ((h0, hS0, rm0), (h1, hS1, rm1)):
            eb = jnp.right_shift(lax.bitcast_convert_type(rm, _i32), 23)
            sc = lax.bitcast_convert_type(jnp.left_shift(254 - eb, 23), _f32)
            for j in range(K):
                off = pl.ds(pl.multiple_of(j * 2048 + lcol, 16), 16)
                hP[off] = hP[off] * sc
            hS[pl.ds(lcol, 16)] = (hS[pl.ds(lcol, 16)] + hS[pl.ds(rcol, 16)]
                                   + (eb - 127).astype(_f32) * np.float32(LN2))
        nxt = gi + 4

        @pl.when(nxt < NCHUNKS)
        def _():
            _wn_start(nxt, par)
        return carry

    with jax.named_scope("chunks0"):
        lax.fori_loop(0, 64, chunk0_body, 0)
    with jax.named_scope("chunks"):
        lax.fori_loop(64, NCHUNKS, chunk_body, 0)

    pltpu.sync_copy(wnsm.at[cid], wsmb)
    m0mask = iot == 0

    def tail_body(s, carry):
        g0 = s < 8
        g1 = s < 12
        g2 = s < 14
        bs = jnp.where(g0, 0, jnp.where(g1, 8, jnp.where(g2, 12, 14)))
        f2s = jnp.where(g0, 8, jnp.where(g1, 4, jnp.where(g2, 2, 1)))
        lcol = s - bs
        rcol = lcol + f2s
        lsp = jnp.broadcast_to(lcol, (16,)).astype(_i32)
        rsp = jnp.broadcast_to(rcol, (16,)).astype(_i32)
        lidx = iot * 2048 + lsp
        ridx = iot * 2048 + rsp
        e0b[...] = plsc.load_gather(h0, [lidx]) * plsc.load_gather(h0, [ridx])
        e1b[...] = plsc.load_gather(h1, [lidx]) * plsc.load_gather(h1, [ridx])
        kk = jnp.broadcast_to(0, (16,)).astype(_i32)
        w = wsmb[s, 0, :]
        a0 = plsc.load_gather(e0b, [kk]) * w
        a1 = plsc.load_gather(e1b, [kk]) * w
        for k in range(1, K):
            kk = jnp.broadcast_to(k, (16,)).astype(_i32)
            w = wsmb[s, k, :]
            a0 = a0 + plsc.load_gather(e0b, [kk]) * w
            a1 = a1 + plsc.load_gather(e1b, [kk]) * w
        mx0 = jnp.broadcast_to(jnp.max(a0), (16,))
        mx1 = jnp.broadcast_to(jnp.max(a1), (16,))
        eb0 = jnp.right_shift(lax.bitcast_convert_type(mx0, _i32), 23)
        eb1 = jnp.right_shift(lax.bitcast_convert_type(mx1, _i32), 23)
        sc0 = lax.bitcast_convert_type(jnp.left_shift(254 - eb0, 23), _f32)
        sc1 = lax.bitcast_convert_type(jnp.left_shift(254 - eb1, 23), _f32)
        plsc.store_scatter(h0, [lidx], a0 * sc0)
        plsc.store_scatter(h1, [lidx], a1 * sc1)
        s0 = (plsc.load_gather(hS0, [lsp]) + plsc.load_gather(hS0, [rsp])
              + (eb0 - 127).astype(_f32) * np.float32(LN2))
        s1 = (plsc.load_gather(hS1, [lsp]) + plsc.load_gather(hS1, [rsp])
              + (eb1 - 127).astype(_f32) * np.float32(LN2))
        plsc.store_scatter(hS0, [lsp], s0, mask=m0mask)
        plsc.store_scatter(hS1, [lsp], s1, mask=m0mask)
        return carry

    with jax.named_scope("tail"):
        lax.fori_loop(0, NSMALL, tail_body, 0)

    z16 = jnp.broadcast_to(0, (16,)).astype(_i32)
    obuf[...] = plsc.load_gather(hS0, [z16]) + _ln(plsc.load_gather(h0, [iot * 2048]))
    pltpu.sync_copy(obuf, out.at[b0])
    obuf[...] = plsc.load_gather(hS1, [z16]) + _ln(plsc.load_gather(h1, [iot * 2048]))
    pltpu.sync_copy(obuf, out.at[b0 + 1])


_sc_call = functools.partial(
    pl.kernel,
    out_type=(
        jax.ShapeDtypeStruct((B, 16), _f32),
        jax.ShapeDtypeStruct((NC, NCHUNKS, K, K, 16), _f32),
        jax.ShapeDtypeStruct((NC, NSMALL, K, K), _f32),
    ),
    mesh=plsc.VectorSubcoreMesh(
        core_axis_name="c", subcore_axis_name="s", num_cores=NC, num_subcores=NS),
    compiler_params=pltpu.CompilerParams(
        needs_layout_passes=False, use_tc_tiling_on_sc=False),
    scratch_types=(
        pltpu.VMEM((K * V,), _f32),
        pltpu.VMEM((K * V,), _f32),
        pltpu.VMEM((V,), _f32),
        pltpu.VMEM((V,), _f32),
        pltpu.VMEM((2, V), _f32),
        pltpu.VMEM((2, 4096), _f32),
        pltpu.VMEM((2, 4096), _f32),
        pltpu.VMEM((V,), _i32),
        pltpu.VMEM((4, K, K, 16), _f32),
        pltpu.VMEM((2, 16, 256), _f32),
        pltpu.VMEM((K, K, 16), _f32),
        pltpu.VMEM((256,), _f32),
        pltpu.VMEM((K, K), _f32),
        pltpu.VMEM((NSMALL, K, K), _f32),
        pltpu.VMEM((16,), _f32),
        pltpu.VMEM((16,), _f32),
        pltpu.VMEM((16,), _f32),
        pltpu.VMEM((256,), _f32),
        pltpu.VMEM((256,), _f32),
        pltpu.SemaphoreType.DMA((2,)),
        pltpu.SemaphoreType.DMA((4,)),
        pltpu.SemaphoreType.DMA((2,)),
    ),
)(_sc_body)

_BRTAB_NP = _bitrev_table(V)


def kernel(x, mu, logsig, w00, w01, w02, w03, w04, w05, w06, w07, w08, w09, w10):
    ws = [w00, w01, w02, w03, w04, w05, w06, w07, w08, w09, w10]
    xp = x.reshape(B, V)
    muf = mu.reshape(V * K)
    lsf = logsig.reshape(V * K)
    wflat = jnp.concatenate([ws[l].reshape(-1, 256) for l in range(NBIG)], axis=0)
    wsm_parts = [ws[l].reshape(-1, 256) for l in range(NBIG, DEPTH - 1)]
    wsm_parts.append(jnp.broadcast_to(w10.reshape(1, 1, K), (1, K, K)).reshape(1, 256))
    wsmf = jnp.concatenate(wsm_parts, axis=0)
    out_wide, _, _ = _sc_call(xp, muf, lsf, wflat, wsmf, jnp.asarray(_BRTAB_NP))
    return out_wide[:, :1]


# speedup vs baseline: 1.0225x; 1.0123x over previous
import functools

import numpy as np
import jax
import jax.numpy as jnp
from jax import lax
from jax.experimental import pallas as pl
from jax.experimental.pallas import tpu as pltpu
from jax.experimental.pallas import tpu_sc as plsc

V = 2048
B = 64
K = 16
DEPTH = 11
NC = 2
NS = 16
NBIG = 7
NCHUNKS = 127
NSMALL = 15
LN2 = 0.6931471805599453
NEG_HALF_LOG_2PI = -0.9189385332046727

_f32 = jnp.float32
_i32 = jnp.int32


def _bitrev_table(n: int) -> np.ndarray:
    bits = n.bit_length() - 1
    idx = np.arange(n)
    rev = np.zeros(n, dtype=np.int32)
    for b in range(bits):
        rev |= ((idx >> b) & 1) << (bits - 1 - b)
    return rev.astype(np.int32)


def _rev11(v):
    r = jnp.zeros_like(v)
    for b in range(11):
        r = jnp.bitwise_or(r, jnp.left_shift(jnp.bitwise_and(jnp.right_shift(v, b), 1), 10 - b))
    return r


def _tree_max(vs):
    while len(vs) > 1:
        vs = [jnp.maximum(vs[i], vs[i + 1]) for i in range(0, len(vs) - 1, 2)] \
            + ([vs[-1]] if len(vs) % 2 else [])
    return vs[0]


def _tree_sum(vs):
    while len(vs) > 1:
        vs = [vs[i] + vs[i + 1] for i in range(0, len(vs) - 1, 2)] \
            + ([vs[-1]] if len(vs) % 2 else [])
    return vs[0]


def _ln(v):
    i = lax.bitcast_convert_type(v, _i32)
    ef = (jnp.right_shift(i, 23) - 127).astype(_f32)
    m = lax.bitcast_convert_type(
        jnp.bitwise_or(jnp.bitwise_and(i, 0x007FFFFF), 0x3F800000), _f32)
    r = (m - 1.0) / (m + 1.0)
    r2 = r * r
    p = 1.0 + r2 * (np.float32(1.0 / 3.0) + r2 * (np.float32(0.2) + r2 * np.float32(1.0 / 7.0)))
    return ef * np.float32(LN2) + 2.0 * r * p


def _sc_body(xp, muf, lsf, wflat, wsmf, brtab, out, wnbig, wnsm, invf,
             h0, h1, hS0, hS1, xb, mub, lsb, invb, btb, wnb, wpa, wpb, wsa, wsb,
             wsmb, e0b, e1b, obuf, msem, wsem, psem, isem):
    cid = lax.axis_index("c")
    sid = lax.axis_index("s")
    wid = cid * NS + sid
    b0 = wid * 2
    iot = lax.iota(_i32, 16)

    pltpu.sync_copy(brtab, btb)

    def _mls_start(seg, par):
        pltpu.async_copy(muf.at[pl.ds(seg * 4096, 4096)], mub.at[par], msem.at[par])
        pltpu.async_copy(lsf.at[pl.ds(seg * 4096, 4096)], lsb.at[par], msem.at[par])
        pltpu.async_copy(invf.at[cid, pl.ds(seg * 4096, 4096)], invb.at[par],
                         msem.at[par])
        pltpu.async_copy(xp.at[b0, pl.ds(seg * 256, 256)], xb.at[par, 0],
                         msem.at[par])
        pltpu.async_copy(xp.at[b0 + 1, pl.ds(seg * 256, 256)], xb.at[par, 1],
                         msem.at[par])

    def _mls_wait(seg, par):
        pltpu.make_async_copy(muf.at[pl.ds(seg * 4096, 4096)], mub.at[par], msem.at[par]).wait()
        pltpu.make_async_copy(lsf.at[pl.ds(seg * 4096, 4096)], lsb.at[par], msem.at[par]).wait()
        pltpu.make_async_copy(invf.at[cid, pl.ds(seg * 4096, 4096)], invb.at[par],
                              msem.at[par]).wait()
        pltpu.make_async_copy(xp.at[b0, pl.ds(seg * 256, 256)], xb.at[par, 0],
                              msem.at[par]).wait()
        pltpu.make_async_copy(xp.at[b0 + 1, pl.ds(seg * 256, 256)], xb.at[par, 1],
                              msem.at[par]).wait()

    scope = jax.named_scope

    def _rows_of(gi):
        l = ((gi >= 64).astype(_i32) + (gi >= 96) + (gi >= 112)
             + (gi >= 120) + (gi >= 124) + (gi >= 126))
        base = jnp.int32(128) - (jnp.int32(128) >> l)
        c = gi - base
        s16 = c * 16 + iot
        f = jnp.right_shift(_rev11(s16), 1 + l)
        return (jnp.int32(2048) - (jnp.int32(2048) >> l)) + f

    def _prep_start(gi, pp):
        pltpu.async_copy(wflat.at[_rows_of(gi)], wpa.at[pp], psem.at[pp])

    def _prep_wait(gi, pp):
        pltpu.make_async_copy(wflat.at[_rows_of(gi)], wpa.at[pp], psem.at[pp]).wait()

    def _prep_chunk(gi, pp):
        ppv = jnp.broadcast_to(pp, (16,)).astype(_i32)

        def jbody(j, carry):
            cols = [jnp.broadcast_to(j * 16 + k, (16,)).astype(_i32)
                    for k in range(K)]
            m = _tree_max([plsc.load_gather(wpa, [ppv, iot, cols[k]])
                           for k in range(K)])
            es = [jnp.exp(plsc.load_gather(wpa, [ppv, iot, cols[k]]) - m)
                  for k in range(K)]
            inv = 1.0 / _tree_sum(es)
            for k in range(K):
                wpb[k, j, :] = es[k] * inv
            return carry

        lax.fori_loop(0, K, jbody, 0)
        pltpu.sync_copy(wpb, wnbig.at[cid, gi])

    @pl.when(sid < NCHUNKS)
    def _():
        _prep_start(sid, 0)

    @pl.when(sid + NS < NCHUNKS)
    def _():
        _prep_start(sid + NS, 1)

    def pbody(i, carry):
        gi = sid + NS * i
        pp = lax.rem(i, 2)

        @pl.when(gi < NCHUNKS)
        def _():
            _prep_wait(gi, pp)
            _prep_chunk(gi, pp)

        @pl.when(gi + 2 * NS < NCHUNKS)
        def _():
            _prep_start(gi + 2 * NS, pp)
        return carry

    with scope("wprep"):
        lax.fori_loop(0, 8, pbody, 0)

    with scope("invprep"):
        ibase = sid * 2048
        pltpu.async_copy(lsf.at[pl.ds(ibase, 2048)], h0.at[pl.ds(0, 2048)], isem).wait()

        def ivbody(q, carry):
            qq = pl.multiple_of(q * 16, 16)
            h0[pl.ds(qq, 16)] = jnp.exp(-h0[pl.ds(qq, 16)])
            return carry

        lax.fori_loop(0, 128, ivbody, 0)
        pltpu.sync_copy(h0.at[pl.ds(0, 2048)], invf.at[cid, pl.ds(ibase, 2048)])

    @pl.when(sid < NSMALL)
    def _():
        s = sid
        g0 = s < 8
        g1 = s < 12
        g2 = s < 14
        bs = jnp.where(g0, 0, jnp.where(g1, 8, jnp.where(g2, 12, 14)))
        fl = s - bs
        r3 = (jnp.left_shift(jnp.bitwise_and(fl, 1), 2)
              | jnp.bitwise_and(fl, 2) | jnp.bitwise_and(jnp.right_shift(fl, 2), 1))
        r2 = jnp.left_shift(jnp.bitwise_and(fl, 1), 1) | jnp.right_shift(fl, 1)
        row = bs + jnp.where(g0, r3, jnp.where(g1, r2, fl))
        pltpu.async_copy(wsmf.at[row], wsa, psem.at[0]).wait()
        m = _tree_max([plsc.load_gather(wsa, [iot * 16 + k]) for k in range(K)])
        es = [jnp.exp(plsc.load_gather(wsa, [iot * 16 + k]) - m)
              for k in range(K)]
        inv = 1.0 / _tree_sum(es)
        for k in range(K):
            wsb[k, :] = es[k] * inv
        pltpu.sync_copy(wsb, wnsm.at[cid, sid])

    with jax.named_scope("barrier"):
        plsc.subcore_barrier()

    _mls_start(0, 0)
    _mls_start(1, 1)

    inscope = jax.named_scope("input_layer")
    inscope.__enter__()
    for seg in range(8):
        par = seg % 2
        _mls_wait(seg, par)
        mu_s = mub.at[par]
        ls_s = lsb.at[par]
        iv_s = invb.at[par]

        def cbody(c, carry2, seg=seg, par=par, mu_s=mu_s, ls_s=ls_s, iv_s=iv_s):
            col = pl.multiple_of(seg * 256 + c * 16, 16)
            cbase = c * 256
            bc = btb[pl.ds(col, 16)]
            cc = pl.multiple_of(c * 16, 16)
            xv0 = xb[par, 0, pl.ds(cc, 16)]
            xv1 = xb[par, 1, pl.ds(cc, 16)]
            for k in range(K):
                idx = iot * 16 + (cbase + k)
                muv = plsc.load_gather(mu_s, [idx])
                lsv = plsc.load_gather(ls_s, [idx])
                inv = plsc.load_gather(iv_s, [idx])
                ct = np.float32(NEG_HALF_LOG_2PI) - lsv
                z0 = (xv0 - muv) * inv
                z1 = (xv1 - muv) * inv
                plsc.store_scatter(h0, [bc + k * 2048], ct - 0.5 * (z0 * z0))
                plsc.store_scatter(h1, [bc + k * 2048], ct - 0.5 * (z1 * z1))
            return carry2

        lax.fori_loop(0, 16, cbody, 0)
        if seg + 2 < 8:
            _mls_start(seg + 2, par)

    inscope.__exit__(None, None, None)
    def _wn_start(gi, par):
        pltpu.async_copy(wnbig.at[cid, gi], wnb.at[par], wsem.at[par])

    def _wn_wait(gi, par):
        pltpu.make_async_copy(wnbig.at[cid, gi], wnb.at[par], wsem.at[par]).wait()

    for q in range(4):
        _wn_start(q, q)

    def chunk0_body(gi, carry):
        par = lax.rem(gi, 4)
        lcol = pl.multiple_of(gi * 16, 16)
        rcol = pl.multiple_of(1024 + gi * 16, 16)
        _wn_wait(gi, par)
        p0 = [h0[pl.ds(pl.multiple_of(k * 2048 + lcol, 16), 16)]
              + h0[pl.ds(pl.multiple_of(k * 2048 + rcol, 16), 16)]
              for k in range(K)]
        p1 = [h1[pl.ds(pl.multiple_of(k * 2048 + lcol, 16), 16)]
              + h1[pl.ds(pl.multiple_of(k * 2048 + rcol, 16), 16)]
              for k in range(K)]
        m0 = _tree_max(p0)
        m1 = _tree_max(p1)
        e0 = [jnp.exp(p0[k] - m0) for k in range(K)]
        e1 = [jnp.exp(p1[k] - m1) for k in range(K)]
        rm0 = None
        rm1 = None
        for j in range(K):
            w = wnb[par, 0, j, :]
            a0 = e0[0] * w
            a1 = e1[0] * w
            for k in range(1, K):
                w = wnb[par, k, j, :]
                a0 = a0 + e0[k] * w
                a1 = a1 + e1[k] * w
            h0[pl.ds(pl.multiple_of(j * 2048 + lcol, 16), 16)] = a0
            h1[pl.ds(pl.multiple_of(j * 2048 + lcol, 16), 16)] = a1
            rm0 = a0 if rm0 is None else jnp.maximum(rm0, a0)
            rm1 = a1 if rm1 is None else jnp.maximum(rm1, a1)
        for hP, hS, rm, m in ((h0, hS0, rm0, m0), (h1, hS1, rm1, m1)):
            eb = jnp.right_shift(lax.bitcast_convert_type(rm, _i32), 23)
            sc = lax.bitcast_convert_type(jnp.left_shift(254 - eb, 23), _f32)
            for j in range(K):
                off = pl.ds(pl.multiple_of(j * 2048 + lcol, 16), 16)
                hP[off] = hP[off] * sc
            hS[pl.ds(lcol, 16)] = m + (eb - 127).astype(_f32) * np.float32(LN2)
        nxt = gi + 4

        @pl.when(nxt < NCHUNKS)
        def _():
            _wn_start(nxt, par)
        return carry

    def chunk_body(gi, carry):
        l = ((gi >= 64).astype(_i32) + (gi >= 96) + (gi >= 112)
             + (gi >= 120) + (gi >= 124) + (gi >= 126))
        f2 = jnp.int32(1024) >> l
        base = jnp.int32(128) - (jnp.int32(128) >> l)
        c = gi - base
        par = lax.rem(gi, 4)
        lcol = pl.multiple_of(c * 16, 16)
        rcol = pl.multiple_of(f2 + c * 16, 16)
        _wn_wait(gi, par)
        e0 = [h0[pl.ds(pl.multiple_of(k * 2048 + lcol, 16), 16)]
              * h0[pl.ds(pl.multiple_of(k * 2048 + rcol, 16), 16)]
              for k in range(K)]
        e1 = [h1[pl.ds(pl.multiple_of(k * 2048 + lcol, 16), 16)]
              * h1[pl.ds(pl.multiple_of(k * 2048 + rcol, 16), 16)]
              for k in range(K)]
        rm0 = None
        rm1 = None
        for j in range(K):
            w = wnb[par, 0, j, :]
            a0 = e0[0] * w
            a1 = e1[0] * w
            for k in range(1, K):
                w = wnb[par, k, j, :]
                a0 = a0 + e0[k] * w
                a1 = a1 + e1[k] * w
            h0[pl.ds(pl.multiple_of(j * 2048 + lcol, 16), 16)] = a0
            h1[pl.ds(pl.multiple_of(j * 2048 + lcol, 16), 16)] = a1
            rm0 = a0 if rm0 is None else jnp.maximum(rm0, a0)
            rm1 = a1 if rm1 is None else jnp.maximum(rm1, a1)
        for hP, hS, rm in ((h0, hS0, rm0), (h1, hS1, rm1)):
            eb = jnp.right_shift(lax.bitcast_convert_type(rm, _i32), 23)
            sc = lax.bitcast_convert_type(jnp.left_shift(254 - eb, 23), _f32)
            for j in range(K):
                off = pl.ds(pl.multiple_of(j * 2048 + lcol, 16), 16)
                hP[off] = hP[off] * sc
            hS[pl.ds(lcol, 16)] = (hS[pl.ds(lcol, 16)] + hS[pl.ds(rcol, 16)]
                                   + (eb - 127).astype(_f32) * np.float32(LN2))
        nxt = gi + 4

        @pl.when(nxt < NCHUNKS)
        def _():
            _wn_start(nxt, par)
        return carry

    with jax.named_scope("chunks0"):
        lax.fori_loop(0, 64, chunk0_body, 0)
    with jax.named_scope("chunks"):
        lax.fori_loop(64, NCHUNKS, chunk_body, 0)

    pltpu.sync_copy(wnsm.at[cid], wsmb)
    m0mask = iot == 0

    def tail_body(s, carry):
        g0 = s < 8
        g1 = s < 12
        g2 = s < 14
        bs = jnp.where(g0, 0, jnp.where(g1, 8, jnp.where(g2, 12, 14)))
        f2s = jnp.where(g0, 8, jnp.where(g1, 4, jnp.where(g2, 2, 1)))
        lcol = s - bs
        rcol = lcol + f2s
        lsp = jnp.broadcast_to(lcol, (16,)).astype(_i32)
        rsp = jnp.broadcast_to(rcol, (16,)).astype(_i32)
        lidx = iot * 2048 + lsp
        ridx = iot * 2048 + rsp
        e0b[...] = plsc.load_gather(h0, [lidx]) * plsc.load_gather(h0, [ridx])
        e1b[...] = plsc.load_gather(h1, [lidx]) * plsc.load_gather(h1, [ridx])
        kk = jnp.broadcast_to(0, (16,)).astype(_i32)
        w = wsmb[s, 0, :]
        a0 = plsc.load_gather(e0b, [kk]) * w
        a1 = plsc.load_gather(e1b, [kk]) * w
        for k in range(1, K):
            kk = jnp.broadcast_to(k, (16,)).astype(_i32)
            w = wsmb[s, k, :]
            a0 = a0 + plsc.load_gather(e0b, [kk]) * w
            a1 = a1 + plsc.load_gather(e1b, [kk]) * w
        mx0 = jnp.broadcast_to(jnp.max(a0), (16,))
        mx1 = jnp.broadcast_to(jnp.max(a1), (16,))
        eb0 = jnp.right_shift(lax.bitcast_convert_type(mx0, _i32), 23)
        eb1 = jnp.right_shift(lax.bitcast_convert_type(mx1, _i32), 23)
        sc0 = lax.bitcast_convert_type(jnp.left_shift(254 - eb0, 23), _f32)
        sc1 = lax.bitcast_convert_type(jnp.left_shift(254 - eb1, 23), _f32)
        plsc.store_scatter(h0, [lidx], a0 * sc0)
        plsc.store_scatter(h1, [lidx], a1 * sc1)
        s0 = (plsc.load_gather(hS0, [lsp]) + plsc.load_gather(hS0, [rsp])
              + (eb0 - 127).astype(_f32) * np.float32(LN2))
        s1 = (plsc.load_gather(hS1, [lsp]) + plsc.load_gather(hS1, [rsp])
              + (eb1 - 127).astype(_f32) * np.float32(LN2))
        plsc.store_scatter(hS0, [lsp], s0, mask=m0mask)
        plsc.store_scatter(hS1, [lsp], s1, mask=m0mask)
        return carry

    with jax.named_scope("tail"):
        lax.fori_loop(0, NSMALL, tail_body, 0)

    z16 = jnp.broadcast_to(0, (16,)).astype(_i32)
    obuf[...] = plsc.load_gather(hS0, [z16]) + _ln(plsc.load_gather(h0, [iot * 2048]))
    pltpu.sync_copy(obuf, out.at[b0])
    obuf[...] = plsc.load_gather(hS1, [z16]) + _ln(plsc.load_gather(h1, [iot * 2048]))
    pltpu.sync_copy(obuf, out.at[b0 + 1])


_sc_call = functools.partial(
    pl.kernel,
    out_type=(
        jax.ShapeDtypeStruct((B, 16), _f32),
        jax.ShapeDtypeStruct((NC, NCHUNKS, K, K, 16), _f32),
        jax.ShapeDtypeStruct((NC, NSMALL, K, K), _f32),
        jax.ShapeDtypeStruct((NC, V * K), _f32),
    ),
    mesh=plsc.VectorSubcoreMesh(
        core_axis_name="c", subcore_axis_name="s", num_cores=NC, num_subcores=NS),
    compiler_params=pltpu.CompilerParams(
        needs_layout_passes=False, use_tc_tiling_on_sc=False),
    scratch_types=(
        pltpu.VMEM((K * V,), _f32),
        pltpu.VMEM((K * V,), _f32),
        pltpu.VMEM((V,), _f32),
        pltpu.VMEM((V,), _f32),
        pltpu.VMEM((2, 2, 256), _f32),
        pltpu.VMEM((2, 4096), _f32),
        pltpu.VMEM((2, 4096), _f32),
        pltpu.VMEM((2, 4096), _f32),
        pltpu.VMEM((V,), _i32),
        pltpu.VMEM((4, K, K, 16), _f32),
        pltpu.VMEM((2, 16, 256), _f32),
        pltpu.VMEM((K, K, 16), _f32),
        pltpu.VMEM((256,), _f32),
        pltpu.VMEM((K, K), _f32),
        pltpu.VMEM((NSMALL, K, K), _f32),
        pltpu.VMEM((16,), _f32),
        pltpu.VMEM((16,), _f32),
        pltpu.VMEM((16,), _f32),
        pltpu.SemaphoreType.DMA((2,)),
        pltpu.SemaphoreType.DMA((4,)),
        pltpu.SemaphoreType.DMA((2,)),
        pltpu.SemaphoreType.DMA,
    ),
)(_sc_body)

_BRTAB_NP = _bitrev_table(V)


def kernel(x, mu, logsig, w00, w01, w02, w03, w04, w05, w06, w07, w08, w09, w10):
    ws = [w00, w01, w02, w03, w04, w05, w06, w07, w08, w09, w10]
    xp = x.reshape(B, V)
    muf = mu.reshape(V * K)
    lsf = logsig.reshape(V * K)
    wflat = jnp.concatenate([ws[l].reshape(-1, 256) for l in range(NBIG)], axis=0)
    wsm_parts = [ws[l].reshape(-1, 256) for l in range(NBIG, DEPTH - 1)]
    wsm_parts.append(jnp.broadcast_to(w10.reshape(1, 1, K), (1, K, K)).reshape(1, 256))
    wsmf = jnp.concatenate(wsm_parts, axis=0)
    out_wide, _, _, _ = _sc_call(xp, muf, lsf, wflat, wsmf, jnp.asarray(_BRTAB_NP))
    return out_wide[:, :1]


# speedup vs baseline: 1.1577x; 1.1322x over previous
import functools

import numpy as np
import jax
import jax.numpy as jnp
from jax import lax
from jax.experimental import pallas as pl
from jax.experimental.pallas import tpu as pltpu
from jax.experimental.pallas import tpu_sc as plsc

V = 2048
B = 64
K = 16
DEPTH = 11
NC = 2
NS = 16
NBIG = 7
NCHUNKS = 127
NSMALL = 15
LN2 = 0.6931471805599453
NEG_HALF_LOG_2PI = -0.9189385332046727

_f32 = jnp.float32
_i32 = jnp.int32


def _bitrev_table(n: int) -> np.ndarray:
    bits = n.bit_length() - 1
    idx = np.arange(n)
    rev = np.zeros(n, dtype=np.int32)
    for b in range(bits):
        rev |= ((idx >> b) & 1) << (bits - 1 - b)
    return rev.astype(np.int32)


def _rev11(v):
    r = jnp.zeros_like(v)
    for b in range(11):
        r = jnp.bitwise_or(r, jnp.left_shift(jnp.bitwise_and(jnp.right_shift(v, b), 1), 10 - b))
    return r


def _ln(v):
    i = lax.bitcast_convert_type(v, _i32)
    ef = (jnp.right_shift(i, 23) - 127).astype(_f32)
    m = lax.bitcast_convert_type(
        jnp.bitwise_or(jnp.bitwise_and(i, 0x007FFFFF), 0x3F800000), _f32)
    r = (m - 1.0) / (m + 1.0)
    r2 = r * r
    p = 1.0 + r2 * (np.float32(1.0 / 3.0) + r2 * (np.float32(0.2) + r2 * np.float32(1.0 / 7.0)))
    return ef * np.float32(LN2) + 2.0 * r * p


def _sc_body(xp, muf, lsf, wflat, wsmf, brtab, out, wnbig, wnsm,
             h0, h1, hS0, hS1, xb, mub, lsb, btb, wnb, wpa, wpb, wsa, wsb,
             wsmb, e0b, e1b, obuf, msem, wsem, psem):
    cid = lax.axis_index("c")
    sid = lax.axis_index("s")
    wid = cid * NS + sid
    b0 = wid * 2
    iot = lax.iota(_i32, 16)

    pltpu.sync_copy(xp.at[b0], xb.at[0])
    pltpu.sync_copy(xp.at[b0 + 1], xb.at[1])
    pltpu.sync_copy(brtab, btb)

    def _mls_start(seg, par):
        pltpu.async_copy(muf.at[pl.ds(seg * 4096, 4096)], mub.at[par], msem.at[par])
        pltpu.async_copy(lsf.at[pl.ds(seg * 4096, 4096)], lsb.at[par], msem.at[par])

    def _mls_wait(seg, par):
        pltpu.make_async_copy(muf.at[pl.ds(seg * 4096, 4096)], mub.at[par], msem.at[par]).wait()
        pltpu.make_async_copy(lsf.at[pl.ds(seg * 4096, 4096)], lsb.at[par], msem.at[par]).wait()

    _mls_start(0, 0)
    _mls_start(1, 1)

    def _rows_of(gi):
        l = ((gi >= 64).astype(_i32) + (gi >= 96) + (gi >= 112)
             + (gi >= 120) + (gi >= 124) + (gi >= 126))
        base = jnp.int32(128) - (jnp.int32(128) >> l)
        c = gi - base
        s16 = c * 16 + iot
        f = jnp.right_shift(_rev11(s16), 1 + l)
        return (jnp.int32(2048) - (jnp.int32(2048) >> l)) + f

    def _prep_start(gi, pp):
        pltpu.async_copy(wflat.at[_rows_of(gi)], wpa.at[pp], psem.at[pp])

    def _prep_wait(gi, pp):
        pltpu.make_async_copy(wflat.at[_rows_of(gi)], wpa.at[pp], psem.at[pp]).wait()

    def _prep_chunk(gi, pp):
        ppv = jnp.broadcast_to(pp, (16,)).astype(_i32)

        def jbody(j, carry):
            rowsk = [plsc.load_gather(
                wpa, [ppv, iot, jnp.broadcast_to(j * 16 + k, (16,)).astype(_i32)])
                for k in range(K)]
            m = rowsk[0]
            for k in range(1, K):
                m = jnp.maximum(m, rowsk[k])
            es = [jnp.exp(rowsk[k] - m) for k in range(K)]
            s = es[0]
            for k in range(1, K):
                s = s + es[k]
            inv = 1.0 / s
            for k in range(K):
                wpb[k, j, :] = es[k] * inv
            return carry

        lax.fori_loop(0, K, jbody, 0)
        pltpu.sync_copy(wpb, wnbig.at[cid, gi])

    @pl.when(sid < NCHUNKS)
    def _():
        _prep_start(sid, 0)

    @pl.when(sid + NS < NCHUNKS)
    def _():
        _prep_start(sid + NS, 1)

    def pbody(i, carry):
        gi = sid + NS * i
        pp = lax.rem(i, 2)

        @pl.when(gi < NCHUNKS)
        def _():
            _prep_wait(gi, pp)
            _prep_chunk(gi, pp)

        @pl.when(gi + 2 * NS < NCHUNKS)
        def _():
            _prep_start(gi + 2 * NS, pp)
        return carry

    lax.fori_loop(0, 8, pbody, 0)

    @pl.when(sid < NSMALL)
    def _():
        s = sid
        g0 = s < 8
        g1 = s < 12
        g2 = s < 14
        bs = jnp.where(g0, 0, jnp.where(g1, 8, jnp.where(g2, 12, 14)))
        fl = s - bs
        r3 = (jnp.left_shift(jnp.bitwise_and(fl, 1), 2)
              | jnp.bitwise_and(fl, 2) | jnp.bitwise_and(jnp.right_shift(fl, 2), 1))
        r2 = jnp.left_shift(jnp.bitwise_and(fl, 1), 1) | jnp.right_shift(fl, 1)
        row = bs + jnp.where(g0, r3, jnp.where(g1, r2, fl))
        pltpu.async_copy(wsmf.at[row], wsa, psem.at[0]).wait()
        rowsk = [plsc.load_gather(wsa, [iot * 16 + k]) for k in range(K)]
        m = rowsk[0]
        for k in range(1, K):
            m = jnp.maximum(m, rowsk[k])
        es = [jnp.exp(rowsk[k] - m) for k in range(K)]
        ssum = es[0]
        for k in range(1, K):
            ssum = ssum + es[k]
        inv = 1.0 / ssum
        for k in range(K):
            wsb[k, :] = es[k] * inv
        pltpu.sync_copy(wsb, wnsm.at[cid, sid])

    for seg in range(8):
        par = seg % 2
        _mls_wait(seg, par)
        mu_s = mub.at[par]
        ls_s = lsb.at[par]

        def cbody(c, carry2, seg=seg, mu_s=mu_s, ls_s=ls_s):
            col = pl.multiple_of(seg * 256 + c * 16, 16)
            cbase = c * 256
            xv0 = xb[0, pl.ds(col, 16)]
            xv1 = xb[1, pl.ds(col, 16)]
            bc = btb[pl.ds(col, 16)]
            ll0 = []
            ll1 = []
            for k in range(K):
                idx = iot * 16 + (cbase + k)
                muv = plsc.load_gather(mu_s, [idx])
                lsv = plsc.load_gather(ls_s, [idx])
                inv = jnp.exp(-lsv)
                ct = np.float32(NEG_HALF_LOG_2PI) - lsv
                z0 = (xv0 - muv) * inv
                z1 = (xv1 - muv) * inv
                ll0.append(ct - 0.5 * (z0 * z0))
                ll1.append(ct - 0.5 * (z1 * z1))
            s0 = ll0[0]
            s1 = ll1[0]
            for k in range(1, K):
                s0 = jnp.maximum(s0, ll0[k])
                s1 = jnp.maximum(s1, ll1[k])
            plsc.store_scatter(hS0, [bc], s0)
            plsc.store_scatter(hS1, [bc], s1)
            for k in range(K):
                plsc.store_scatter(h0, [bc + k * 2048], jnp.exp(ll0[k] - s0))
                plsc.store_scatter(h1, [bc + k * 2048], jnp.exp(ll1[k] - s1))
            return carry2

        lax.fori_loop(0, 16, cbody, 0)
        if seg + 2 < 8:
            _mls_start(seg + 2, par)

    plsc.subcore_barrier()

    def _wn_start(gi, par):
        pltpu.async_copy(wnbig.at[cid, gi], wnb.at[par], wsem.at[par])

    def _wn_wait(gi, par):
        pltpu.make_async_copy(wnbig.at[cid, gi], wnb.at[par], wsem.at[par]).wait()

    for q in range(4):
        _wn_start(q, q)

    def chunk_body(gi, carry):
        l = ((gi >= 64).astype(_i32) + (gi >= 96) + (gi >= 112)
             + (gi >= 120) + (gi >= 124) + (gi >= 126))
        f2 = jnp.int32(1024) >> l
        base = jnp.int32(128) - (jnp.int32(128) >> l)
        c = gi - base
        par = lax.rem(gi, 4)
        lcol = pl.multiple_of(c * 16, 16)
        rcol = pl.multiple_of(f2 + c * 16, 16)
        _wn_wait(gi, par)
        e0 = [h0[pl.ds(pl.multiple_of(k * 2048 + lcol, 16), 16)] * h0[pl.ds(pl.multiple_of(k * 2048 + rcol, 16), 16)]
              for k in range(K)]
        e1 = [h1[pl.ds(pl.multiple_of(k * 2048 + lcol, 16), 16)] * h1[pl.ds(pl.multiple_of(k * 2048 + rcol, 16), 16)]
              for k in range(K)]
        rm0 = None
        rm1 = None
        for j in range(K):
            w = wnb[par, 0, j, :]
            a0 = e0[0] * w
            a1 = e1[0] * w
            for k in range(1, K):
                w = wnb[par, k, j, :]
                a0 = a0 + e0[k] * w
                a1 = a1 + e1[k] * w
            h0[pl.ds(pl.multiple_of(j * 2048 + lcol, 16), 16)] = a0
            h1[pl.ds(pl.multiple_of(j * 2048 + lcol, 16), 16)] = a1
            rm0 = a0 if rm0 is None else jnp.maximum(rm0, a0)
            rm1 = a1 if rm1 is None else jnp.maximum(rm1, a1)
        eb0 = jnp.right_shift(lax.bitcast_convert_type(rm0, _i32), 23)
        eb1 = jnp.right_shift(lax.bitcast_convert_type(rm1, _i32), 23)
        sc0 = lax.bitcast_convert_type(jnp.left_shift(254 - eb0, 23), _f32)
        sc1 = lax.bitcast_convert_type(jnp.left_shift(254 - eb1, 23), _f32)
        for j in range(K):
            h0[pl.ds(pl.multiple_of(j * 2048 + lcol, 16), 16)] = h0[pl.ds(pl.multiple_of(j * 2048 + lcol, 16), 16)] * sc0
            h1[pl.ds(pl.multiple_of(j * 2048 + lcol, 16), 16)] = h1[pl.ds(pl.multiple_of(j * 2048 + lcol, 16), 16)] * sc1
        hS0[pl.ds(lcol, 16)] = (hS0[pl.ds(lcol, 16)] + hS0[pl.ds(rcol, 16)]
                                + (eb0 - 127).astype(_f32) * np.float32(LN2))
        hS1[pl.ds(lcol, 16)] = (hS1[pl.ds(lcol, 16)] + hS1[pl.ds(rcol, 16)]
                                + (eb1 - 127).astype(_f32) * np.float32(LN2))
        nxt = gi + 4

        @pl.when(nxt < NCHUNKS)
        def _():
            _wn_start(nxt, par)
        return carry

    lax.fori_loop(0, NCHUNKS, chunk_body, 0)

    pltpu.sync_copy(wnsm.at[cid], wsmb)
    m0mask = iot == 0

    def tail_body(s, carry):
        g0 = s < 8
        g1 = s < 12
        g2 = s < 14
        bs = jnp.where(g0, 0, jnp.where(g1, 8, jnp.where(g2, 12, 14)))
        f2s = jnp.where(g0, 8, jnp.where(g1, 4, jnp.where(g2, 2, 1)))
        lcol = s - bs
        rcol = lcol + f2s
        lsp = jnp.broadcast_to(lcol, (16,)).astype(_i32)
        rsp = jnp.broadcast_to(rcol, (16,)).astype(_i32)
        lidx = iot * 2048 + lsp
        ridx = iot * 2048 + rsp
        e0b[...] = plsc.load_gather(h0, [lidx]) * plsc.load_gather(h0, [ridx])
        e1b[...] = plsc.load_gather(h1, [lidx]) * plsc.load_gather(h1, [ridx])
        kk = jnp.broadcast_to(0, (16,)).astype(_i32)
        w = wsmb[s, 0, :]
        a0 = plsc.load_gather(e0b, [kk]) * w
        a1 = plsc.load_gather(e1b, [kk]) * w
        for k in range(1, K):
            kk = jnp.broadcast_to(k, (16,)).astype(_i32)
            w = wsmb[s, k, :]
            a0 = a0 + plsc.load_gather(e0b, [kk]) * w
            a1 = a1 + plsc.load_gather(e1b, [kk]) * w
        mx0 = jnp.broadcast_to(jnp.max(a0), (16,))
        mx1 = jnp.broadcast_to(jnp.max(a1), (16,))
        eb0 = jnp.right_shift(lax.bitcast_convert_type(mx0, _i32), 23)
        eb1 = jnp.right_shift(lax.bitcast_convert_type(mx1, _i32), 23)
        sc0 = lax.bitcast_convert_type(jnp.left_shift(254 - eb0, 23), _f32)
        sc1 = lax.bitcast_convert_type(jnp.left_shift(254 - eb1, 23), _f32)
        plsc.store_scatter(h0, [lidx], a0 * sc0)
        plsc.store_scatter(h1, [lidx], a1 * sc1)
        s0 = (plsc.load_gather(hS0, [lsp]) + plsc.load_gather(hS0, [rsp])
              + (eb0 - 127).astype(_f32) * np.float32(LN2))
        s1 = (plsc.load_gather(hS1, [lsp]) + plsc.load_gather(hS1, [rsp])
              + (eb1 - 127).astype(_f32) * np.float32(LN2))
        plsc.store_scatter(hS0, [lsp], s0, mask=m0mask)
        plsc.store_scatter(hS1, [lsp], s1, mask=m0mask)
        return carry

    lax.fori_loop(0, NSMALL, tail_body, 0)

    z16 = jnp.broadcast_to(0, (16,)).astype(_i32)
    obuf[...] = plsc.load_gather(hS0, [z16]) + _ln(plsc.load_gather(h0, [iot * 2048]))
    pltpu.sync_copy(obuf, out.at[b0])
    obuf[...] = plsc.load_gather(hS1, [z16]) + _ln(plsc.load_gather(h1, [iot * 2048]))
    pltpu.sync_copy(obuf, out.at[b0 + 1])


_sc_call = functools.partial(
    pl.kernel,
    out_type=(
        jax.ShapeDtypeStruct((B, 16), _f32),
        jax.ShapeDtypeStruct((NC, NCHUNKS, K, K, 16), _f32),
        jax.ShapeDtypeStruct((NC, NSMALL, K, K), _f32),
    ),
    mesh=plsc.VectorSubcoreMesh(
        core_axis_name="c", subcore_axis_name="s", num_cores=NC, num_subcores=NS),
    compiler_params=pltpu.CompilerParams(
        needs_layout_passes=False, use_tc_tiling_on_sc=False),
    scratch_types=(
        pltpu.VMEM((K * V,), _f32),
        pltpu.VMEM((K * V,), _f32),
        pltpu.VMEM((V,), _f32),
        pltpu.VMEM((V,), _f32),
        pltpu.VMEM((2, V), _f32),
        pltpu.VMEM((2, 4096), _f32),
        pltpu.VMEM((2, 4096), _f32),
        pltpu.VMEM((V,), _i32),
        pltpu.VMEM((4, K, K, 16), _f32),
        pltpu.VMEM((2, 16, 256), _f32),
        pltpu.VMEM((K, K, 16), _f32),
        pltpu.VMEM((256,), _f32),
        pltpu.VMEM((K, K), _f32),
        pltpu.VMEM((NSMALL, K, K), _f32),
        pltpu.VMEM((16,), _f32),
        pltpu.VMEM((16,), _f32),
        pltpu.VMEM((16,), _f32),
        pltpu.SemaphoreType.DMA((2,)),
        pltpu.SemaphoreType.DMA((4,)),
        pltpu.SemaphoreType.DMA((2,)),
    ),
)(_sc_body)

_BRTAB_NP = _bitrev_table(V)


def kernel(x, mu, logsig, w00, w01, w02, w03, w04, w05, w06, w07, w08, w09, w10):
    ws = [w00, w01, w02, w03, w04, w05, w06, w07, w08, w09, w10]
    xp = x.reshape(B, V)
    muf = mu.reshape(V * K)
    lsf = logsig.reshape(V * K)
    wflat = jnp.concatenate([ws[l].reshape(-1, 256) for l in range(NBIG)], axis=0)
    wsm_parts = [ws[l].reshape(-1, 256) for l in range(NBIG, DEPTH - 1)]
    wsm_parts.append(jnp.broadcast_to(w10.reshape(1, 1, K), (1, K, K)).reshape(1, 256))
    wsmf = jnp.concatenate(wsm_parts, axis=0)
    out_wide, _, _ = _sc_call(xp, muf, lsf, wflat, wsmf, jnp.asarray(_BRTAB_NP))
    return out_wide[:, :1]
